# Initial kernel scaffold; baseline (speedup 1.0000x reference)
import jax
import jax.numpy as jnp
from jax.experimental import pallas as pl


def kernel(token_ids, weight):
    raise NotImplementedError("write your pallas kernel here")


# speedup vs baseline: 1.6840x; 1.6840x over previous
import functools

import jax
import jax.numpy as jnp
from jax import lax
from jax.experimental import pallas as pl
from jax.experimental.pallas import tpu as pltpu
from jax.experimental.pallas import tpu_sc as plsc

_NUM_CORES = 2
_NUM_SUBCORES = 16
_NW = _NUM_CORES * _NUM_SUBCORES
_G = 128


def _make_sc_gather(B, D, n_chunks):
    mesh = plsc.VectorSubcoreMesh(core_axis_name="c", subcore_axis_name="s")

    @functools.partial(
        pl.kernel,
        mesh=mesh,
        out_type=jax.ShapeDtypeStruct((B, D), jnp.float32),
        scratch_types=[
            pltpu.VMEM((n_chunks, _G), jnp.int32),
            pltpu.VMEM((_G, D), jnp.float32),
            pltpu.SemaphoreType.DMA,
        ],
        compiler_params=pltpu.CompilerParams(use_tc_tiling_on_sc=False),
    )
    def body(idx_hbm, tab_hbm, out_hbm, idx_v, rows_v, sem):
        wid = lax.axis_index("s") * _NUM_CORES + lax.axis_index("c")
        base = wid * (n_chunks * _G)
        pltpu.sync_copy(idx_hbm.at[wid], idx_v)

        def step(g, carry):
            pltpu.async_copy(tab_hbm.at[idx_v.at[g]], rows_v, sem).wait()
            pltpu.sync_copy(rows_v, out_hbm.at[pl.ds(base + g * _G, _G)])
            return carry

        lax.fori_loop(0, n_chunks, step, 0)

    return body


def kernel(token_ids, weight):
    Bt, S = token_ids.shape
    V, D = weight.shape
    B = Bt * S
    n_chunks = B // (_NW * _G)
    idx = token_ids.astype(jnp.int32).reshape(_NW, n_chunks, _G)
    out = _make_sc_gather(B, D, n_chunks)(idx, weight)
    return out.reshape(Bt, S, D)


# speedup vs baseline: 1.8755x; 1.1137x over previous
import functools

import jax
import jax.numpy as jnp
from jax import lax
from jax.experimental import pallas as pl
from jax.experimental.pallas import tpu as pltpu
from jax.experimental.pallas import tpu_sc as plsc

_NUM_CORES = 2
_NUM_SUBCORES = 16
_NW = _NUM_CORES * _NUM_SUBCORES
_G = 128
_NB = 8
_K = 5


def _make_sc_gather(B, D, n_chunks):
    mesh = plsc.VectorSubcoreMesh(core_axis_name="c", subcore_axis_name="s")

    @functools.partial(
        pl.kernel,
        mesh=mesh,
        out_type=jax.ShapeDtypeStruct((B, D), jnp.float32),
        scratch_types=[
            pltpu.VMEM((n_chunks, _G), jnp.int32),
            pltpu.VMEM((_NB, _G, D), jnp.float32),
            pltpu.SemaphoreType.DMA((_NB,)),
            pltpu.SemaphoreType.DMA((_NB,)),
        ],
        compiler_params=pltpu.CompilerParams(use_tc_tiling_on_sc=False),
    )
    def body(idx_hbm, tab_hbm, out_hbm, idx_v, rows_v, gsem, ssem):
        wid = lax.axis_index("s") * _NUM_CORES + lax.axis_index("c")
        base = wid * (n_chunks * _G)
        pltpu.sync_copy(idx_hbm.at[wid], idx_v)

        def gather_start(g, b):
            pltpu.async_copy(tab_hbm.at[idx_v.at[g]], rows_v.at[b], gsem.at[b])

        def gather_wait(g, b):
            pltpu.make_async_copy(
                tab_hbm.at[idx_v.at[g]], rows_v.at[b], gsem.at[b]).wait()

        def store_start(g, b):
            pltpu.async_copy(
                rows_v.at[b], out_hbm.at[pl.ds(base + g * _G, _G)], ssem.at[b])

        def store_wait(g, b):
            pltpu.make_async_copy(
                rows_v.at[b], out_hbm.at[pl.ds(base + g * _G, _G)],
                ssem.at[b]).wait()

        for b in range(_K):
            gather_start(b, b)

        def outer(i, carry):
            g0 = i * _NB
            for b in range(_NB):
                g = g0 + b
                bf = (g + _K) % _NB
                @pl.when(g + _K < n_chunks)
                def _():
                    @pl.when(g + _K - _NB >= 0)
                    def _():
                        store_wait(g + _K - _NB, bf)
                    gather_start(g + _K, bf)

                gather_wait(g, b)
                store_start(g, b)
            return carry

        lax.fori_loop(0, n_chunks // _NB, outer, 0)

        for j in range(_NB):
            g = n_chunks - _NB + j
            store_wait(g, g % _NB)

    return body


def kernel(token_ids, weight):
    Bt, S = token_ids.shape
    V, D = weight.shape
    B = Bt * S
    n_chunks = B // (_NW * _G)
    idx = token_ids.astype(jnp.int32).reshape(_NW, n_chunks, _G)
    out = _make_sc_gather(B, D, n_chunks)(idx, weight)
    return out.reshape(Bt, S, D)
